# baseline (speedup 1.0000x reference)
import functools

import jax
import jax.numpy as jnp
from jax import lax
from jax.experimental import pallas as pl
from jax.experimental.pallas import tpu as pltpu
from jax.experimental.pallas import tpu_sc as plsc

N = 10000
E = 320000
D = 128
G = 144
H1 = 8
C1 = 16

NC = 2
NS = 16
NW = NC * NS
EPW = E // NW
KB = 80
NBLK = EPW // KB
NBT = E // KB
RPS = 624
TAIL = N - RPS * NS
TOFF = RPS * NS

_f32 = jnp.float32


def _proj_body(x_ref, wg_ref, vd_ref, g_ref, ad_ref):
    xb = x_ref[...]
    g_ref[...] = jnp.dot(xb, wg_ref[...], preferred_element_type=_f32)
    ad_ref[...] = jnp.dot(xb, vd_ref[...], preferred_element_type=_f32)


def _proj(x, wg, vd, blk=1000):
    d = x.shape[1]
    grid = (N // blk,)
    return pl.pallas_call(
        _proj_body,
        grid=grid,
        in_specs=[
            pl.BlockSpec((blk, d), lambda i: (i, 0)),
            pl.BlockSpec((d, G), lambda i: (0, 0)),
            pl.BlockSpec((d, 16), lambda i: (0, 0)),
        ],
        out_specs=[
            pl.BlockSpec((blk, G), lambda i: (i, 0)),
            pl.BlockSpec((blk, 16), lambda i: (i, 0)),
        ],
        out_shape=[
            jax.ShapeDtypeStruct((N, G), _f32),
            jax.ShapeDtypeStruct((N, 16), _f32),
        ],
    )(x, wg, vd)


def _ptab_body(g_ref, ad_ref, p_ref):
    a_s = g_ref[:, D:G]
    a_d = ad_ref[...]
    m = a_d + jnp.max(a_s, axis=0, keepdims=True)
    m = jnp.maximum(m, 0.2 * m)
    p_ref[...] = jnp.concatenate([a_d[:, 0:8], -m[:, 0:8]], axis=1)


def _ptab(g, a_d):
    return pl.pallas_call(
        _ptab_body,
        out_shape=jax.ShapeDtypeStruct((N, 16), _f32),
    )(g, a_d)


def _mid_body(acc_ref, e1_ref, b1_ref, wg2_ref, vd2_ref, g2_ref, ad2_ref):
    accg = acc_ref[0] + acc_ref[1]
    acc = accg[:, 0:D]
    den = accg[:, D:G]
    den_e = jnp.dot(den, e1_ref[...], preferred_element_type=_f32)
    h = acc / (den_e + 1e-16) + b1_ref[...]
    h = jnp.maximum(h, 0.0)
    g2_ref[...] = jnp.dot(h, wg2_ref[...], preferred_element_type=_f32)
    ad2_ref[...] = jnp.dot(h, vd2_ref[...], preferred_element_type=_f32)


def _mid(acc, e1, b1, wg2, vd2, blk=1000):
    grid = (N // blk,)
    return pl.pallas_call(
        _mid_body,
        grid=grid,
        in_specs=[
            pl.BlockSpec((2, blk, G), lambda i: (0, i, 0)),
            pl.BlockSpec((16, D), lambda i: (0, 0)),
            pl.BlockSpec((1, D), lambda i: (0, 0)),
            pl.BlockSpec((D, G), lambda i: (0, 0)),
            pl.BlockSpec((D, 16), lambda i: (0, 0)),
        ],
        out_specs=[
            pl.BlockSpec((blk, G), lambda i: (i, 0)),
            pl.BlockSpec((blk, 16), lambda i: (i, 0)),
        ],
        out_shape=[
            jax.ShapeDtypeStruct((N, G), _f32),
            jax.ShapeDtypeStruct((N, 16), _f32),
        ],
    )(acc, e1, b1, wg2, vd2)


def _final_body(acc_ref, e2_ref, b2_ref, out_ref):
    accg = acc_ref[0] + acc_ref[1]
    acc = accg[:, 0:D]
    den = accg[:, D:G]
    den_e = jnp.dot(den, e2_ref[...], preferred_element_type=_f32)
    out_ref[...] = acc / (den_e + 1e-16) + b2_ref[...]


def _final(acc, e2, b2, blk=1000):
    grid = (N // blk,)
    return pl.pallas_call(
        _final_body,
        grid=grid,
        in_specs=[
            pl.BlockSpec((2, blk, G), lambda i: (0, i, 0)),
            pl.BlockSpec((16, D), lambda i: (0, 0)),
            pl.BlockSpec((1, D), lambda i: (0, 0)),
        ],
        out_specs=pl.BlockSpec((blk, D), lambda i: (i, 0)),
        out_shape=jax.ShapeDtypeStruct((N, D), _f32),
    )(acc, e2, b2)


def _edge_pass_body(ei_hbm, g_hbm, p_hbm, z_hbm, acc_out,
                    acc_sh, idx_v, g_v, p_v,
                    semg0, semg1, semg2, semp0, semp1, semp2,
                    sems0, sems1, sems2):
    cid = lax.axis_index("c")
    sid = lax.axis_index("s")
    wid = sid * NC + cid
    semg = [semg0, semg1, semg2]
    semp = [semp0, semp1, semp2]
    sems = [sems0, sems1, sems2]

    pltpu.sync_copy(z_hbm.at[pl.ds(sid * RPS, RPS)],
                    acc_sh.at[pl.ds(sid * RPS, RPS)])

    @pl.when(sid == 0)
    def _tail_zero():
        pltpu.sync_copy(z_hbm.at[pl.ds(TOFF, TAIL)],
                        acc_sh.at[pl.ds(TOFF, TAIL)])

    plsc.subcore_barrier()

    iota16 = lax.iota(jnp.int32, 16)
    shuf_lo = lax.rem(iota16, jnp.full((16,), 8, jnp.int32))
    shuf_hi = shuf_lo + jnp.full((16,), 8, jnp.int32)
    lane8 = iota16 < jnp.full((16,), 8, jnp.int32)
    zero16 = jnp.zeros((16,), _f32)

    def fire(ph, b):
        pltpu.sync_copy(ei_hbm.at[wid * NBLK + b], idx_v.at[ph])
        pltpu.async_copy(g_hbm.at[idx_v.at[ph, 0]], g_v.at[ph], semg[ph])
        pltpu.async_copy(p_hbm.at[idx_v.at[ph, 1]], p_v.at[ph], semp[ph])

    def wait_g(ph):
        pltpu.make_async_copy(g_hbm.at[idx_v.at[ph, 0]], g_v.at[ph],
                              semg[ph]).wait()
        pltpu.make_async_copy(p_hbm.at[idx_v.at[ph, 1]], p_v.at[ph],
                              semp[ph]).wait()

    def scat(ph):
        pltpu.async_copy(g_v.at[ph], acc_sh.at[idx_v.at[ph, 1]], sems[ph],
                         add=True)

    def wait_scat(ph):
        pltpu.make_async_copy(g_v.at[ph], acc_sh.at[idx_v.at[ph, 1]],
                              sems[ph]).wait()

    def compute(ph):
        @plsc.parallel_loop(0, KB, unroll=12)
        def _edge(i):
            a_s = g_v[ph, i, D:G]
            p16 = p_v[ph, i, :]
            t = a_s + p16
            msh = jnp.take(t, shuf_hi)
            alpha = jnp.maximum(t, 0.2 * t)
            ex = jnp.exp(alpha + msh)
            ex = jnp.where(lane8, ex, zero16)
            g_v[ph, i, D:G] = ex
            spl = jnp.take(ex, shuf_lo)
            for j in range(8):
                sl = pl.ds(j * 16, 16)
                g_v[ph, i, sl] = g_v[ph, i, sl] * spl

    fire(0, 0)
    fire(1, 1)

    @pl.loop(0, NBLK - 2, step=3)
    def _blk(b):
        for s in range(3):
            ph = s
            nph = (s + 2) % 3
            bb = b + s
            wait_g(ph)
            compute(ph)
            scat(ph)

            @pl.when(bb >= 1)
            def _ws():
                wait_scat(nph)

            fire(nph, bb + 2)

    wait_g(0)
    compute(0)
    scat(0)
    wait_g(1)
    compute(1)
    scat(1)
    wait_scat(2)
    wait_scat(0)
    wait_scat(1)

    plsc.subcore_barrier()
    pltpu.sync_copy(acc_sh.at[pl.ds(sid * RPS, RPS)],
                    acc_out.at[cid, pl.ds(sid * RPS, RPS)])

    @pl.when(sid == 0)
    def _tail_dump():
        pltpu.sync_copy(acc_sh.at[pl.ds(TOFF, TAIL)],
                        acc_out.at[cid, pl.ds(TOFF, TAIL)])


def _edge_pass(ei, g, p, z):
    mesh = plsc.VectorSubcoreMesh(core_axis_name="c", subcore_axis_name="s")
    f = pl.kernel(
        _edge_pass_body,
        compiler_params=pltpu.CompilerParams(use_tc_tiling_on_sc=False),
        out_type=jax.ShapeDtypeStruct((NC, N, G), _f32),
        mesh=mesh,
        scratch_types=[
            pltpu.VMEM_SHARED((N, G), _f32),
            pltpu.VMEM((3, 2, KB), jnp.int32),
            pltpu.VMEM((3, KB, G), _f32),
            pltpu.VMEM((3, KB, 16), _f32),
        ] + [pltpu.SemaphoreType.DMA] * 9,
    )
    return f(ei, g, p, z)


def kernel(x, edge_index, W_src1, W_dst1, att_src1, att_dst1, b1,
           W_src2, W_dst2, att_src2, att_dst2, b2):
    ei = edge_index.astype(jnp.int32).reshape(2, NBT, KB).transpose(1, 0, 2)

    idx = (jnp.arange(D) % H1) * C1 + (jnp.arange(D) // H1)

    a1s = (att_src1[:, :, None] * jnp.eye(H1, dtype=_f32)[:, None, :]).reshape(D, H1)
    a1d = (att_dst1[:, :, None] * jnp.eye(H1, dtype=_f32)[:, None, :]).reshape(D, H1)
    vs1 = jnp.pad(W_src1 @ a1s, ((0, 0), (0, 8)))
    vd1 = jnp.pad(W_dst1 @ a1d, ((0, 0), (0, 8)))
    wg1 = jnp.concatenate([W_src1[:, idx], vs1], axis=1)

    w2p = W_src2[idx, :]
    v2s = jnp.pad(jnp.tile((w2p @ att_src2[0])[:, None], (1, 8)),
                  ((0, 0), (0, 8)))
    v2d = jnp.tile((W_dst2[idx, :] @ att_dst2[0])[:, None], (1, 16))
    wg2 = jnp.concatenate([w2p, v2s], axis=1)
    b1p = b1[idx][None, :]
    b2r = b2[None, :]

    e1 = (jnp.arange(16)[:, None] == (jnp.arange(D)[None, :] % H1)).astype(_f32)
    e2 = (jnp.arange(16)[:, None] == 0).astype(_f32) * jnp.ones((1, D), _f32)

    z = jnp.zeros((N, G), _f32)

    g1, ad1 = _proj(x, wg1, vd1)
    p1 = _ptab(g1, ad1)
    acc1 = _edge_pass(ei, g1, p1, z)

    g2, ad2 = _mid(acc1, e1, b1p, wg2, v2d)
    p2 = _ptab(g2, ad2)
    acc2 = _edge_pass(ei, g2, p2, z)

    return _final(acc2, e2, b2r)


# speedup vs baseline: 1.0819x; 1.0819x over previous
import functools

import jax
import jax.numpy as jnp
from jax import lax
from jax.experimental import pallas as pl
from jax.experimental.pallas import tpu as pltpu
from jax.experimental.pallas import tpu_sc as plsc

N = 10000
E = 320000
D = 128
G = 144
H1 = 8
C1 = 16

NC = 2
NS = 16
NW = NC * NS
EPW = E // NW
KB = 80
NBLK = EPW // KB
NBT = E // KB
RPS = 624
TAIL = N - RPS * NS
TOFF = RPS * NS

_f32 = jnp.float32


def _proj_body(x_ref, wg_ref, vd_ref, g_ref, ad_ref):
    xb = x_ref[...].astype(jnp.bfloat16)
    g_ref[...] = jnp.dot(xb, wg_ref[...].astype(jnp.bfloat16),
                         preferred_element_type=_f32)
    ad_ref[...] = jnp.dot(xb, vd_ref[...].astype(jnp.bfloat16),
                          preferred_element_type=_f32)


def _proj(x, wg, vd, blk=1000):
    d = x.shape[1]
    grid = (N // blk,)
    return pl.pallas_call(
        _proj_body,
        grid=grid,
        in_specs=[
            pl.BlockSpec((blk, d), lambda i: (i, 0)),
            pl.BlockSpec((d, G), lambda i: (0, 0)),
            pl.BlockSpec((d, 16), lambda i: (0, 0)),
        ],
        out_specs=[
            pl.BlockSpec((blk, G), lambda i: (i, 0)),
            pl.BlockSpec((blk, 16), lambda i: (i, 0)),
        ],
        out_shape=[
            jax.ShapeDtypeStruct((N, G), _f32),
            jax.ShapeDtypeStruct((N, 16), _f32),
        ],
    )(x, wg, vd)


def _ptab_body(g_ref, ad_ref, p_ref):
    a_s = g_ref[:, D:G]
    a_d = ad_ref[...]
    m = a_d + jnp.max(a_s, axis=0, keepdims=True)
    m = jnp.maximum(m, 0.2 * m)
    p_ref[...] = jnp.concatenate([a_d[:, 0:8], -m[:, 0:8]], axis=1)


def _ptab(g, a_d):
    return pl.pallas_call(
        _ptab_body,
        out_shape=jax.ShapeDtypeStruct((N, 16), _f32),
    )(g, a_d)


def _mid_body(acc_ref, e1_ref, b1_ref, wg2_ref, vd2_ref, g2_ref, ad2_ref):
    accg = acc_ref[0] + acc_ref[1]
    acc = accg[:, 0:D]
    den = accg[:, D:G]
    den_e = jnp.dot(den, e1_ref[...], preferred_element_type=_f32)
    h = acc / (den_e + 1e-16) + b1_ref[...]
    h = jnp.maximum(h, 0.0)
    h16 = h.astype(jnp.bfloat16)
    g2_ref[...] = jnp.dot(h16, wg2_ref[...].astype(jnp.bfloat16),
                          preferred_element_type=_f32)
    ad2_ref[...] = jnp.dot(h16, vd2_ref[...].astype(jnp.bfloat16),
                           preferred_element_type=_f32)


def _mid(acc, e1, b1, wg2, vd2, blk=1000):
    grid = (N // blk,)
    return pl.pallas_call(
        _mid_body,
        grid=grid,
        in_specs=[
            pl.BlockSpec((2, blk, G), lambda i: (0, i, 0)),
            pl.BlockSpec((16, D), lambda i: (0, 0)),
            pl.BlockSpec((1, D), lambda i: (0, 0)),
            pl.BlockSpec((D, G), lambda i: (0, 0)),
            pl.BlockSpec((D, 16), lambda i: (0, 0)),
        ],
        out_specs=[
            pl.BlockSpec((blk, G), lambda i: (i, 0)),
            pl.BlockSpec((blk, 16), lambda i: (i, 0)),
        ],
        out_shape=[
            jax.ShapeDtypeStruct((N, G), _f32),
            jax.ShapeDtypeStruct((N, 16), _f32),
        ],
    )(acc, e1, b1, wg2, vd2)


def _final_body(acc_ref, e2_ref, b2_ref, out_ref):
    accg = acc_ref[0] + acc_ref[1]
    acc = accg[:, 0:D]
    den = accg[:, D:G]
    den_e = jnp.dot(den, e2_ref[...], preferred_element_type=_f32)
    out_ref[...] = acc / (den_e + 1e-16) + b2_ref[...]


def _final(acc, e2, b2, blk=1000):
    grid = (N // blk,)
    return pl.pallas_call(
        _final_body,
        grid=grid,
        in_specs=[
            pl.BlockSpec((2, blk, G), lambda i: (0, i, 0)),
            pl.BlockSpec((16, D), lambda i: (0, 0)),
            pl.BlockSpec((1, D), lambda i: (0, 0)),
        ],
        out_specs=pl.BlockSpec((blk, D), lambda i: (i, 0)),
        out_shape=jax.ShapeDtypeStruct((N, D), _f32),
    )(acc, e2, b2)


def _edge_pass_body(ei_hbm, g_hbm, p_hbm, acc_out,
                    acc_sh, idx_v, g_v, p_v,
                    semg0, semg1, semg2, semp0, semp1, semp2,
                    sems0, sems1, sems2):
    cid = lax.axis_index("c")
    sid = lax.axis_index("s")
    wid = sid * NC + cid
    semg = [semg0, semg1, semg2]
    semp = [semp0, semp1, semp2]
    sems = [sems0, sems1, sems2]

    zrow = jnp.zeros((16,), _f32)

    @plsc.parallel_loop(0, KB, unroll=8)
    def _zb(i):
        for k in range(G // 16):
            g_v[0, i, pl.ds(k * 16, 16)] = zrow

    for k in range(RPS // KB):
        pltpu.sync_copy(g_v.at[0],
                        acc_sh.at[pl.ds(sid * RPS + k * KB, KB)])
    pltpu.sync_copy(g_v.at[0, pl.ds(0, RPS % KB)],
                    acc_sh.at[pl.ds(sid * RPS + (RPS // KB) * KB, RPS % KB)])

    @pl.when(sid == 0)
    def _tail_zero():
        pltpu.sync_copy(g_v.at[0, pl.ds(0, TAIL)],
                        acc_sh.at[pl.ds(TOFF, TAIL)])

    plsc.subcore_barrier()

    iota16 = lax.iota(jnp.int32, 16)
    shuf_lo = lax.rem(iota16, jnp.full((16,), 8, jnp.int32))
    shuf_hi = shuf_lo + jnp.full((16,), 8, jnp.int32)
    lane8 = iota16 < jnp.full((16,), 8, jnp.int32)
    zero16 = jnp.zeros((16,), _f32)

    def fire(ph, b):
        pltpu.sync_copy(ei_hbm.at[wid * NBLK + b], idx_v.at[ph])
        pltpu.async_copy(g_hbm.at[idx_v.at[ph, 0]], g_v.at[ph], semg[ph])
        pltpu.async_copy(p_hbm.at[idx_v.at[ph, 1]], p_v.at[ph], semp[ph])

    def wait_g(ph):
        pltpu.make_async_copy(g_hbm.at[idx_v.at[ph, 0]], g_v.at[ph],
                              semg[ph]).wait()
        pltpu.make_async_copy(p_hbm.at[idx_v.at[ph, 1]], p_v.at[ph],
                              semp[ph]).wait()

    def scat(ph):
        pltpu.async_copy(g_v.at[ph], acc_sh.at[idx_v.at[ph, 1]], sems[ph],
                         add=True)

    def wait_scat(ph):
        pltpu.make_async_copy(g_v.at[ph], acc_sh.at[idx_v.at[ph, 1]],
                              sems[ph]).wait()

    def compute(ph):
        @plsc.parallel_loop(0, KB, unroll=8)
        def _edge(i):
            a_s = g_v[ph, i, D:G]
            p16 = p_v[ph, i, :]
            t = a_s + p16
            msh = jnp.take(t, shuf_hi)
            alpha = jnp.maximum(t, 0.2 * t)
            ex = jnp.exp(alpha + msh)
            ex = jnp.where(lane8, ex, zero16)
            g_v[ph, i, D:G] = ex
            spl = jnp.take(ex, shuf_lo)
            for j in range(8):
                sl = pl.ds(j * 16, 16)
                g_v[ph, i, sl] = g_v[ph, i, sl] * spl

    fire(0, 0)
    fire(1, 1)

    @pl.loop(0, NBLK - 2, step=3)
    def _blk(b):
        for s in range(3):
            ph = s
            nph = (s + 2) % 3
            bb = b + s
            wait_g(ph)
            compute(ph)
            scat(ph)

            @pl.when(bb >= 1)
            def _ws():
                wait_scat(nph)

            fire(nph, bb + 2)

    wait_g(0)
    compute(0)
    scat(0)
    wait_g(1)
    compute(1)
    scat(1)
    wait_scat(2)
    wait_scat(0)
    wait_scat(1)

    plsc.subcore_barrier()
    pltpu.sync_copy(acc_sh.at[pl.ds(sid * RPS, RPS)],
                    acc_out.at[cid, pl.ds(sid * RPS, RPS)])

    @pl.when(sid == 0)
    def _tail_dump():
        pltpu.sync_copy(acc_sh.at[pl.ds(TOFF, TAIL)],
                        acc_out.at[cid, pl.ds(TOFF, TAIL)])


def _edge_pass(ei, g, p):
    mesh = plsc.VectorSubcoreMesh(core_axis_name="c", subcore_axis_name="s")
    f = pl.kernel(
        _edge_pass_body,
        compiler_params=pltpu.CompilerParams(use_tc_tiling_on_sc=False),
        out_type=jax.ShapeDtypeStruct((NC, N, G), _f32),
        mesh=mesh,
        scratch_types=[
            pltpu.VMEM_SHARED((N, G), _f32),
            pltpu.VMEM((3, 2, KB), jnp.int32),
            pltpu.VMEM((3, KB, G), _f32),
            pltpu.VMEM((3, KB, 16), _f32),
        ] + [pltpu.SemaphoreType.DMA] * 9,
    )
    return f(ei, g, p)


def kernel(x, edge_index, W_src1, W_dst1, att_src1, att_dst1, b1,
           W_src2, W_dst2, att_src2, att_dst2, b2):
    ei = edge_index.astype(jnp.int32).reshape(2, NBT, KB).transpose(1, 0, 2)

    idx = (jnp.arange(D) % H1) * C1 + (jnp.arange(D) // H1)

    a1s = (att_src1[:, :, None] * jnp.eye(H1, dtype=_f32)[:, None, :]).reshape(D, H1)
    a1d = (att_dst1[:, :, None] * jnp.eye(H1, dtype=_f32)[:, None, :]).reshape(D, H1)
    vs1 = jnp.pad(W_src1 @ a1s, ((0, 0), (0, 8)))
    vd1 = jnp.pad(W_dst1 @ a1d, ((0, 0), (0, 8)))
    wg1 = jnp.concatenate([W_src1[:, idx], vs1], axis=1)

    w2p = W_src2[idx, :]
    v2s = jnp.pad(jnp.tile((w2p @ att_src2[0])[:, None], (1, 8)),
                  ((0, 0), (0, 8)))
    v2d = jnp.tile((W_dst2[idx, :] @ att_dst2[0])[:, None], (1, 16))
    wg2 = jnp.concatenate([w2p, v2s], axis=1)
    b1p = b1[idx][None, :]
    b2r = b2[None, :]

    e1 = (jnp.arange(16)[:, None] == (jnp.arange(D)[None, :] % H1)).astype(_f32)
    e2 = (jnp.arange(16)[:, None] == 0).astype(_f32) * jnp.ones((1, D), _f32)

    g1, ad1 = _proj(x, wg1, vd1)
    p1 = _ptab(g1, ad1)
    acc1 = _edge_pass(ei, g1, p1)

    g2, ad2 = _mid(acc1, e1, b1p, wg2, v2d)
    p2 = _ptab(g2, ad2)
    acc2 = _edge_pass(ei, g2, p2)

    return _final(acc2, e2, b2r)


# speedup vs baseline: 1.1187x; 1.0341x over previous
import functools

import jax
import jax.numpy as jnp
from jax import lax
from jax.experimental import pallas as pl
from jax.experimental.pallas import tpu as pltpu
from jax.experimental.pallas import tpu_sc as plsc

N = 10000
E = 320000
D = 128
G = 144
H1 = 8
C1 = 16

NC = 2
NS = 16
NW = NC * NS
EPW = E // NW
KB = 80
NBLK = EPW // KB
NBT = E // KB
RPS = 624
TAIL = N - RPS * NS
TOFF = RPS * NS

_f32 = jnp.float32


_NB = 10


def _ptab_tail(p_ref, ad_acc, m_acc):
    a_d = ad_acc[...]
    m = a_d + m_acc[...]
    m = jnp.maximum(m, 0.2 * m)
    p_ref[...] = jnp.concatenate([a_d[:, 0:8], -m[:, 0:8]], axis=1)


def _accum_attn(i, gv, adv, ad_acc, m_acc, blk):
    @pl.when(i == 0)
    def _init():
        m_acc[...] = jnp.full((1, 16), -1e30, _f32)

    ad_acc[pl.ds(i * blk, blk), :] = adv
    m_acc[...] = jnp.maximum(m_acc[...],
                             jnp.max(gv[:, D:G], axis=0, keepdims=True))


def _proj_body(x_ref, wg_ref, vd_ref, g_ref, p_ref, ad_acc, m_acc, blk):
    i = pl.program_id(0)

    @pl.when(i < _NB)
    def _blk():
        xb = x_ref[...].astype(jnp.bfloat16)
        gv = jnp.dot(xb, wg_ref[...].astype(jnp.bfloat16),
                     preferred_element_type=_f32)
        g_ref[...] = gv
        adv = jnp.dot(xb, vd_ref[...].astype(jnp.bfloat16),
                      preferred_element_type=_f32)
        _accum_attn(i, gv, adv, ad_acc, m_acc, blk)

    @pl.when(i == _NB)
    def _tail():
        _ptab_tail(p_ref, ad_acc, m_acc)


def _proj(x, wg, vd, blk=1000):
    d = x.shape[1]
    return pl.pallas_call(
        functools.partial(_proj_body, blk=blk),
        grid=(_NB + 1,),
        in_specs=[
            pl.BlockSpec((blk, d), lambda i: (jnp.minimum(i, _NB - 1), 0)),
            pl.BlockSpec((d, G), lambda i: (0, 0)),
            pl.BlockSpec((d, 16), lambda i: (0, 0)),
        ],
        out_specs=[
            pl.BlockSpec((blk, G), lambda i: (jnp.minimum(i, _NB - 1), 0)),
            pl.BlockSpec((N, 16), lambda i: (0, 0)),
        ],
        out_shape=[
            jax.ShapeDtypeStruct((N, G), _f32),
            jax.ShapeDtypeStruct((N, 16), _f32),
        ],
        scratch_shapes=[
            pltpu.VMEM((N, 16), _f32),
            pltpu.VMEM((1, 16), _f32),
        ],
    )(x, wg, vd)


def _mid_body(acc_ref, e1_ref, b1_ref, wg2_ref, vd2_ref, g2_ref, p_ref,
              ad_acc, m_acc, blk):
    i = pl.program_id(0)

    @pl.when(i < _NB)
    def _blk():
        accg = acc_ref[0] + acc_ref[1]
        acc = accg[:, 0:D]
        den = accg[:, D:G]
        den_e = jnp.dot(den, e1_ref[...], preferred_element_type=_f32)
        h = acc / (den_e + 1e-16) + b1_ref[...]
        h = jnp.maximum(h, 0.0)
        h16 = h.astype(jnp.bfloat16)
        gv = jnp.dot(h16, wg2_ref[...].astype(jnp.bfloat16),
                     preferred_element_type=_f32)
        g2_ref[...] = gv
        adv = jnp.dot(h16, vd2_ref[...].astype(jnp.bfloat16),
                      preferred_element_type=_f32)
        _accum_attn(i, gv, adv, ad_acc, m_acc, blk)

    @pl.when(i == _NB)
    def _tail():
        _ptab_tail(p_ref, ad_acc, m_acc)


def _mid(acc, e1, b1, wg2, vd2, blk=1000):
    return pl.pallas_call(
        functools.partial(_mid_body, blk=blk),
        grid=(_NB + 1,),
        in_specs=[
            pl.BlockSpec((2, blk, G),
                         lambda i: (0, jnp.minimum(i, _NB - 1), 0)),
            pl.BlockSpec((16, D), lambda i: (0, 0)),
            pl.BlockSpec((1, D), lambda i: (0, 0)),
            pl.BlockSpec((D, G), lambda i: (0, 0)),
            pl.BlockSpec((D, 16), lambda i: (0, 0)),
        ],
        out_specs=[
            pl.BlockSpec((blk, G), lambda i: (jnp.minimum(i, _NB - 1), 0)),
            pl.BlockSpec((N, 16), lambda i: (0, 0)),
        ],
        out_shape=[
            jax.ShapeDtypeStruct((N, G), _f32),
            jax.ShapeDtypeStruct((N, 16), _f32),
        ],
        scratch_shapes=[
            pltpu.VMEM((N, 16), _f32),
            pltpu.VMEM((1, 16), _f32),
        ],
    )(acc, e1, b1, wg2, vd2)


def _final_body(acc_ref, e2_ref, b2_ref, out_ref):
    accg = acc_ref[0] + acc_ref[1]
    acc = accg[:, 0:D]
    den = accg[:, D:G]
    den_e = jnp.dot(den, e2_ref[...], preferred_element_type=_f32)
    out_ref[...] = acc / (den_e + 1e-16) + b2_ref[...]


def _final(acc, e2, b2, blk=1000):
    grid = (N // blk,)
    return pl.pallas_call(
        _final_body,
        grid=grid,
        in_specs=[
            pl.BlockSpec((2, blk, G), lambda i: (0, i, 0)),
            pl.BlockSpec((16, D), lambda i: (0, 0)),
            pl.BlockSpec((1, D), lambda i: (0, 0)),
        ],
        out_specs=pl.BlockSpec((blk, D), lambda i: (i, 0)),
        out_shape=jax.ShapeDtypeStruct((N, D), _f32),
    )(acc, e2, b2)


def _edge_pass_body(ei_hbm, g_hbm, p_hbm, acc_out,
                    acc_sh, idx_v, g_v, p_v,
                    semg0, semg1, semg2, semp0, semp1, semp2,
                    sems0, sems1, sems2):
    cid = lax.axis_index("c")
    sid = lax.axis_index("s")
    wid = sid * NC + cid
    semg = [semg0, semg1, semg2]
    semp = [semp0, semp1, semp2]
    sems = [sems0, sems1, sems2]

    zrow = jnp.zeros((16,), _f32)

    @plsc.parallel_loop(0, KB, unroll=8)
    def _zb(i):
        for k in range(G // 16):
            g_v[0, i, pl.ds(k * 16, 16)] = zrow

    for k in range(RPS // KB):
        pltpu.sync_copy(g_v.at[0],
                        acc_sh.at[pl.ds(sid * RPS + k * KB, KB)])
    pltpu.sync_copy(g_v.at[0, pl.ds(0, RPS % KB)],
                    acc_sh.at[pl.ds(sid * RPS + (RPS // KB) * KB, RPS % KB)])

    @pl.when(sid == 0)
    def _tail_zero():
        pltpu.sync_copy(g_v.at[0, pl.ds(0, TAIL)],
                        acc_sh.at[pl.ds(TOFF, TAIL)])

    plsc.subcore_barrier()

    iota16 = lax.iota(jnp.int32, 16)
    shuf_lo = lax.rem(iota16, jnp.full((16,), 8, jnp.int32))
    shuf_hi = shuf_lo + jnp.full((16,), 8, jnp.int32)
    lane8 = iota16 < jnp.full((16,), 8, jnp.int32)
    zero16 = jnp.zeros((16,), _f32)

    def fire(ph, b):
        pltpu.sync_copy(ei_hbm.at[wid * NBLK + b], idx_v.at[ph])
        pltpu.async_copy(g_hbm.at[idx_v.at[ph, 0]], g_v.at[ph], semg[ph])
        pltpu.async_copy(p_hbm.at[idx_v.at[ph, 1]], p_v.at[ph], semp[ph])

    def wait_g(ph):
        pltpu.make_async_copy(g_hbm.at[idx_v.at[ph, 0]], g_v.at[ph],
                              semg[ph]).wait()
        pltpu.make_async_copy(p_hbm.at[idx_v.at[ph, 1]], p_v.at[ph],
                              semp[ph]).wait()

    def scat(ph):
        pltpu.async_copy(g_v.at[ph], acc_sh.at[idx_v.at[ph, 1]], sems[ph],
                         add=True)

    def wait_scat(ph):
        pltpu.make_async_copy(g_v.at[ph], acc_sh.at[idx_v.at[ph, 1]],
                              sems[ph]).wait()

    def compute(ph):
        @plsc.parallel_loop(0, KB, unroll=8)
        def _edge(i):
            a_s = g_v[ph, i, D:G]
            p16 = p_v[ph, i, :]
            t = a_s + p16
            msh = jnp.take(t, shuf_hi)
            alpha = jnp.maximum(t, 0.2 * t)
            ex = jnp.exp(alpha + msh)
            ex = jnp.where(lane8, ex, zero16)
            g_v[ph, i, D:G] = ex
            spl = jnp.take(ex, shuf_lo)
            for j in range(8):
                sl = pl.ds(j * 16, 16)
                g_v[ph, i, sl] = g_v[ph, i, sl] * spl

    fire(0, 0)
    fire(1, 1)

    @pl.loop(0, NBLK - 2, step=3)
    def _blk(b):
        for s in range(3):
            ph = s
            nph = (s + 2) % 3
            bb = b + s
            wait_g(ph)
            compute(ph)
            scat(ph)

            @pl.when(bb >= 1)
            def _ws():
                wait_scat(nph)

            fire(nph, bb + 2)

    wait_g(0)
    compute(0)
    scat(0)
    wait_g(1)
    compute(1)
    scat(1)
    wait_scat(2)
    wait_scat(0)
    wait_scat(1)

    plsc.subcore_barrier()
    pltpu.sync_copy(acc_sh.at[pl.ds(sid * RPS, RPS)],
                    acc_out.at[cid, pl.ds(sid * RPS, RPS)])

    @pl.when(sid == 0)
    def _tail_dump():
        pltpu.sync_copy(acc_sh.at[pl.ds(TOFF, TAIL)],
                        acc_out.at[cid, pl.ds(TOFF, TAIL)])


def _edge_pass(ei, g, p):
    mesh = plsc.VectorSubcoreMesh(core_axis_name="c", subcore_axis_name="s")
    f = pl.kernel(
        _edge_pass_body,
        compiler_params=pltpu.CompilerParams(use_tc_tiling_on_sc=False),
        out_type=jax.ShapeDtypeStruct((NC, N, G), _f32),
        mesh=mesh,
        scratch_types=[
            pltpu.VMEM_SHARED((N, G), _f32),
            pltpu.VMEM((3, 2, KB), jnp.int32),
            pltpu.VMEM((3, KB, G), _f32),
            pltpu.VMEM((3, KB, 16), _f32),
        ] + [pltpu.SemaphoreType.DMA] * 9,
    )
    return f(ei, g, p)


def kernel(x, edge_index, W_src1, W_dst1, att_src1, att_dst1, b1,
           W_src2, W_dst2, att_src2, att_dst2, b2):
    ei = edge_index.astype(jnp.int32).reshape(2, NBT, KB).transpose(1, 0, 2)

    idx = (jnp.arange(D) % H1) * C1 + (jnp.arange(D) // H1)

    a1s = (att_src1[:, :, None] * jnp.eye(H1, dtype=_f32)[:, None, :]).reshape(D, H1)
    a1d = (att_dst1[:, :, None] * jnp.eye(H1, dtype=_f32)[:, None, :]).reshape(D, H1)
    vs1 = jnp.pad(W_src1 @ a1s, ((0, 0), (0, 8)))
    vd1 = jnp.pad(W_dst1 @ a1d, ((0, 0), (0, 8)))
    wg1 = jnp.concatenate([W_src1[:, idx], vs1], axis=1)

    w2p = W_src2[idx, :]
    v2s = jnp.pad(jnp.tile((w2p @ att_src2[0])[:, None], (1, 8)),
                  ((0, 0), (0, 8)))
    v2d = jnp.tile((W_dst2[idx, :] @ att_dst2[0])[:, None], (1, 16))
    wg2 = jnp.concatenate([w2p, v2s], axis=1)
    b1p = b1[idx][None, :]
    b2r = b2[None, :]

    e1 = (jnp.arange(16)[:, None] == (jnp.arange(D)[None, :] % H1)).astype(_f32)
    e2 = (jnp.arange(16)[:, None] == 0).astype(_f32) * jnp.ones((1, D), _f32)

    g1, p1 = _proj(x, wg1, vd1)
    acc1 = _edge_pass(ei, g1, p1)

    g2, p2 = _mid(acc1, e1, b1p, wg2, v2d)
    acc2 = _edge_pass(ei, g2, p2)

    return _final(acc2, e2, b2r)
